# baseline (speedup 1.0000x reference)
import functools

import jax
import jax.numpy as jnp
from jax import lax
from jax.experimental import pallas as pl
from jax.experimental.pallas import tpu as pltpu
from jax.experimental.pallas import tpu_sc as plsc

_LANES = 16


@functools.lru_cache(maxsize=None)
def _build_call(batch, dim, uvocab, ivocab):
    info = plsc.get_sparse_core_info()
    nc, ns = info.num_cores, info.num_subcores
    nw = nc * ns
    assert batch % (8 * nw) == 0
    bpw = batch // nw
    groups = bpw // _LANES

    mesh = plsc.VectorSubcoreMesh(core_axis_name="c", subcore_axis_name="s")

    @functools.partial(
        pl.kernel,
        mesh=mesh,
        compiler_params=pltpu.CompilerParams(
            needs_layout_passes=False, use_tc_tiling_on_sc=False),
        out_type=jax.ShapeDtypeStruct((batch,), jnp.float32),
        scratch_types=[
            pltpu.VMEM((bpw,), jnp.int32),
            pltpu.VMEM((bpw,), jnp.int32),
            pltpu.VMEM((bpw, dim), jnp.float32),
            pltpu.VMEM((bpw, dim), jnp.float32),
            pltpu.VMEM((bpw,), jnp.float32),
            pltpu.VMEM((bpw,), jnp.float32),
            pltpu.VMEM((bpw,), jnp.float32),
            pltpu.SemaphoreType.DMA,
        ],
    )
    def call(user_hbm, item_hbm, uemb_hbm, iemb_hbm, ubias_hbm, ibias_hbm,
             out_hbm, uidx_v, iidx_v, ue_v, ie_v, ub_v, ib_v, out_v, sem):
        wid = lax.axis_index("s") * nc + lax.axis_index("c")
        base = wid * bpw
        pltpu.sync_copy(user_hbm.at[pl.ds(base, bpw)], uidx_v)
        pltpu.sync_copy(item_hbm.at[pl.ds(base, bpw)], iidx_v)
        cp0 = pltpu.async_copy(uemb_hbm.at[uidx_v], ue_v, sem)
        cp1 = pltpu.async_copy(iemb_hbm.at[iidx_v], ie_v, sem)
        cp2 = pltpu.async_copy(ubias_hbm.at[uidx_v], ub_v, sem)
        cp3 = pltpu.async_copy(ibias_hbm.at[iidx_v], ib_v, sem)
        cp0.wait()
        cp1.wait()
        cp2.wait()
        cp3.wait()

        def group(g, carry):
            rows = lax.iota(jnp.int32, _LANES) + g * _LANES
            acc = ub_v[pl.ds(g * _LANES, _LANES)] + ib_v[pl.ds(g * _LANES, _LANES)]
            for d in range(dim):
                col = jnp.full((_LANES,), d, jnp.int32)
                acc = acc + (plsc.load_gather(ue_v, [rows, col])
                             * plsc.load_gather(ie_v, [rows, col]))
            out_v[pl.ds(g * _LANES, _LANES)] = acc
            return carry

        lax.fori_loop(0, groups, group, 0)
        pltpu.sync_copy(out_v, out_hbm.at[pl.ds(base, bpw)])

    return call


def kernel(user, item, user_emb_w, item_emb_w, user_bias_w, item_bias_w):
    batch = user.shape[0]
    uvocab, dim = user_emb_w.shape
    ivocab = item_emb_w.shape[0]
    call = _build_call(batch, dim, uvocab, ivocab)
    return call(user.astype(jnp.int32), item.astype(jnp.int32),
                user_emb_w, item_emb_w,
                user_bias_w.reshape(-1), item_bias_w.reshape(-1))


# speedup vs baseline: 2.8481x; 2.8481x over previous
import functools

import jax
import jax.numpy as jnp
from jax import lax
from jax.experimental import pallas as pl
from jax.experimental.pallas import tpu as pltpu
from jax.experimental.pallas import tpu_sc as plsc

_LANES = 16


def _flat_view(w):
    v, d = w.shape
    nb = v // 128
    t = w.T[:, : nb * 128]
    t = t.reshape(d // 8, 8, nb, 128).transpose(0, 2, 1, 3)
    return t.reshape(-1)


@functools.lru_cache(maxsize=None)
def _build_call(batch, dim, uvocab, ivocab):
    info = plsc.get_sparse_core_info()
    nc, ns = info.num_cores, info.num_subcores
    nw = nc * ns
    assert batch % (8 * nw) == 0 and dim % 8 == 0
    bpw = batch // nw
    groups = bpw // _LANES
    unb, inb = uvocab // 128, ivocab // 128
    utrunc, itrunc = unb * 128, inb * 128
    utail, itail = uvocab - utrunc, ivocab - itrunc

    mesh = plsc.VectorSubcoreMesh(core_axis_name="c", subcore_axis_name="s")

    @functools.partial(
        pl.kernel,
        mesh=mesh,
        compiler_params=pltpu.CompilerParams(
            needs_layout_passes=False, use_tc_tiling_on_sc=False),
        out_type=jax.ShapeDtypeStruct((batch,), jnp.float32),
        scratch_types=[
            pltpu.VMEM((bpw,), jnp.int32),
            pltpu.VMEM((bpw,), jnp.int32),
            pltpu.VMEM((bpw,), jnp.int32),
            pltpu.VMEM((bpw,), jnp.int32),
            pltpu.VMEM((dim, bpw), jnp.float32),
            pltpu.VMEM((dim, bpw), jnp.float32),
            pltpu.VMEM((max(utail, 8), dim), jnp.float32),
            pltpu.VMEM((max(itail, 8), dim), jnp.float32),
            pltpu.VMEM((bpw,), jnp.float32),
            pltpu.VMEM((bpw,), jnp.float32),
            pltpu.VMEM((bpw,), jnp.float32),
            pltpu.SemaphoreType.DMA,
        ],
    )
    def call(user_hbm, item_hbm, uflat_hbm, iflat_hbm, uside_hbm, iside_hbm,
             ubias_hbm, ibias_hbm, out_hbm,
             uidx_v, iidx_v, uoff_v, ioff_v, ue_v, ie_v, uside_v, iside_v,
             ub_v, ib_v, out_v, sem):
        wid = lax.axis_index("s") * nc + lax.axis_index("c")
        base = wid * bpw
        pltpu.sync_copy(user_hbm.at[pl.ds(base, bpw)], uidx_v)
        pltpu.sync_copy(item_hbm.at[pl.ds(base, bpw)], iidx_v)

        def offsets(g, _):
            sl = pl.ds(g * _LANES, _LANES)
            ru = uidx_v[sl]
            ru = jnp.minimum(ru, utrunc - 1)
            uoff_v[sl] = (ru >> 7) * 1024 + (ru & 127)
            ri = iidx_v[sl]
            ri = jnp.minimum(ri, itrunc - 1)
            ioff_v[sl] = (ri >> 7) * 1024 + (ri & 127)
            return _

        lax.fori_loop(0, groups, offsets, 0)

        copies = [
            pltpu.async_copy(uside_hbm, uside_v.at[pl.ds(0, utail)], sem),
            pltpu.async_copy(iside_hbm, iside_v.at[pl.ds(0, itail)], sem),
            pltpu.async_copy(ubias_hbm.at[0].at[uidx_v], ub_v, sem),
            pltpu.async_copy(ibias_hbm.at[0].at[iidx_v], ib_v, sem),
        ]
        for d in range(dim):
            cu = (d // 8) * unb * 1024 + (d % 8) * 128
            span_u = unb * 1024 - (d % 8) * 128
            copies.append(pltpu.async_copy(
                uflat_hbm.at[pl.ds(cu, span_u)].at[uoff_v], ue_v.at[d], sem))
            ci = (d // 8) * inb * 1024 + (d % 8) * 128
            span_i = inb * 1024 - (d % 8) * 128
            copies.append(pltpu.async_copy(
                iflat_hbm.at[pl.ds(ci, span_i)].at[ioff_v], ie_v.at[d], sem))
        for cp in copies:
            cp.wait()

        def group(g, carry):
            sl = pl.ds(g * _LANES, _LANES)
            ru = uidx_v[sl]
            ri = iidx_v[sl]
            umask = ru >= utrunc
            imask = ri >= itrunc
            su = jnp.where(umask, ru - utrunc, 0)
            si = jnp.where(imask, ri - itrunc, 0)
            acc = ub_v[sl] + ib_v[sl]
            for d in range(dim):
                dcol = jnp.full((_LANES,), d, jnp.int32)
                uval = jnp.where(umask, plsc.load_gather(uside_v, [su, dcol]),
                                 ue_v[d, sl])
                ival = jnp.where(imask, plsc.load_gather(iside_v, [si, dcol]),
                                 ie_v[d, sl])
                acc = acc + uval * ival
            out_v[sl] = acc
            return carry

        lax.fori_loop(0, groups, group, 0)
        pltpu.sync_copy(out_v, out_hbm.at[pl.ds(base, bpw)])

    return call


def kernel(user, item, user_emb_w, item_emb_w, user_bias_w, item_bias_w):
    batch = user.shape[0]
    uvocab, dim = user_emb_w.shape
    ivocab = item_emb_w.shape[0]
    call = _build_call(batch, dim, uvocab, ivocab)
    return call(user.astype(jnp.int32), item.astype(jnp.int32),
                _flat_view(user_emb_w), _flat_view(item_emb_w),
                user_emb_w[(uvocab // 128) * 128:, :],
                item_emb_w[(ivocab // 128) * 128:, :],
                user_bias_w.T, item_bias_w.T)
